# Initial kernel scaffold; baseline (speedup 1.0000x reference)
import jax
import jax.numpy as jnp
from jax.experimental import pallas as pl


def kernel(inp, tgt, w0, b0, w1, b1, w2, b2, w3, b3, w4, b4, w5, b5, w6, b6, w7, b7, w8, b8, w9, b9):
    raise NotImplementedError("write your pallas kernel here")


# speedup vs baseline: 4.2332x; 4.2332x over previous
import functools

import numpy as np
import jax
import jax.numpy as jnp
from jax import lax
from jax.experimental import pallas as pl
from jax.experimental.pallas import tpu as pltpu

_MEAN = (0.485, 0.456, 0.406)
_STD = (0.229, 0.224, 0.225)

_VMEM = 60 * 1024 * 1024

_PW = {224: 232, 112: 120, 56: 64, 28: 32, 8: 16}


def _bilinear_matrix(out_size, in_size):
    mat = np.zeros((out_size, in_size), dtype=np.float32)
    scale = in_size / out_size
    for d in range(out_size):
        src = max((d + 0.5) * scale - 0.5, 0.0)
        i0 = min(int(np.floor(src)), in_size - 1)
        i1 = min(i0 + 1, in_size - 1)
        w1 = np.float32(src - i0)
        mat[d, i0] += np.float32(1.0) - w1
        mat[d, i1] += w1
    return jnp.asarray(mat)


def _resize_kernel(x_ref, rh_ref, rwt_ref, o_ref):
    c = lax.rem(pl.program_id(0), 3)
    t = jnp.dot(rh_ref[...], x_ref[0], preferred_element_type=jnp.float32)
    r = jnp.dot(t, rwt_ref[...], preferred_element_type=jnp.float32)
    inv_s = jnp.where(c == 0, np.float32(1.0 / _STD[0]),
                      jnp.where(c == 1, np.float32(1.0 / _STD[1]),
                                np.float32(1.0 / _STD[2])))
    m_s = jnp.where(c == 0, np.float32(_MEAN[0] / _STD[0]),
                    jnp.where(c == 1, np.float32(_MEAN[1] / _STD[1]),
                              np.float32(_MEAN[2] / _STD[2])))
    o_ref[0] = (r * inv_s - m_s).astype(jnp.bfloat16)


def _resize_norm(z, out_hw=224):
    b, c, h, w = z.shape
    rh = _bilinear_matrix(out_hw, h)
    rwt = _bilinear_matrix(out_hw, w).T
    zf = z.reshape(b * c, h, w)
    out = pl.pallas_call(
        _resize_kernel,
        out_shape=jax.ShapeDtypeStruct((b * c, out_hw, out_hw), jnp.bfloat16),
        grid=(b * c,),
        in_specs=[
            pl.BlockSpec((1, h, w), lambda i: (i, 0, 0)),
            pl.BlockSpec((out_hw, h), lambda i: (0, 0)),
            pl.BlockSpec((w, out_hw), lambda i: (0, 0)),
        ],
        out_specs=pl.BlockSpec((1, out_hw, out_hw), lambda i: (i, 0, 0)),
        compiler_params=pltpu.CompilerParams(
            dimension_semantics=("parallel",), vmem_limit_bytes=_VMEM),
    )(zf, rh, rwt)
    return out.reshape(b, c, out_hw, out_hw)


def _c11_kernel(a_ref, w_ref, b_ref, o_ref, *, th, wd, pw, cout):
    acc = jnp.dot(a_ref[...], w_ref[...], preferred_element_type=jnp.float32)
    val = jnp.maximum(acc + b_ref[...], 0.0).astype(jnp.bfloat16)
    r3 = val.reshape(th, wd, cout)
    zl = jnp.zeros((th, 1, cout), jnp.bfloat16)
    zr = jnp.zeros((th, pw - wd - 1, cout), jnp.bfloat16)
    o_ref[0] = jnp.concatenate([zl, r3, zr], axis=1)


def _conv1_1(rz, w, b, th=56):
    bsz, cin, hd, wd = rz.shape
    cout = w.shape[-1]
    pw = _PW[wd]
    xp = jnp.pad(rz, ((0, 0), (0, 0), (1, 1), (1, 1)))
    cols = [xp[:, c, dy:dy + hd, dx:dx + wd]
            for dy in range(3) for dx in range(3) for c in range(cin)]
    a = jnp.stack(cols, axis=-1).reshape(bsz * hd * wd, 9 * cin)
    wf = w.reshape(9 * cin, cout).astype(jnp.bfloat16)
    bf = b.reshape(1, cout)
    nb = hd // th
    body = functools.partial(_c11_kernel, th=th, wd=wd, pw=pw, cout=cout)
    out = pl.pallas_call(
        body,
        out_shape=jax.ShapeDtypeStruct((bsz, hd, pw, cout), jnp.bfloat16),
        grid=(bsz * nb,),
        in_specs=[
            pl.BlockSpec((th * wd, 9 * cin), lambda g: (g, 0)),
            pl.BlockSpec((9 * cin, cout), lambda g: (0, 0)),
            pl.BlockSpec((1, cout), lambda g: (0, 0)),
        ],
        out_specs=pl.BlockSpec((1, th, pw, cout),
                               lambda g, nb=nb: (g // nb, g % nb, 0, 0)),
        compiler_params=pltpu.CompilerParams(
            dimension_semantics=("parallel",), vmem_limit_bytes=_VMEM),
    )(a, wf, bf)
    return out


def _conv_kernel(xm_ref, xh_ref, w_ref, b_ref, o_ref, *, th, pw, wd, c):
    xm = xm_ref[0]
    top = xh_ref[0, 0, 0:1]
    bot = xh_ref[0, 0, 1:2]
    xf = jnp.concatenate([top, xm, bot], axis=0)
    z1 = jnp.zeros((th + 2, 1, c), jnp.bfloat16)
    left = jnp.concatenate([z1, xf[:, :pw - 1]], axis=1)
    right = jnp.concatenate([xf[:, 1:], z1], axis=1)
    a = jnp.concatenate([left, xf, right], axis=2)
    cout = o_ref.shape[-1]
    acc = jnp.zeros((th * pw, cout), jnp.float32)
    for dy in range(3):
        acc = acc + jnp.dot(a[dy:dy + th].reshape(th * pw, 3 * c), w_ref[dy],
                            preferred_element_type=jnp.float32)
    val = jnp.maximum(acc + b_ref[...], 0.0).reshape(th, pw, cout)
    ci = lax.broadcasted_iota(jnp.int32, (th, pw, cout), 1)
    keep = (ci >= 1) & (ci <= wd)
    o_ref[0] = jnp.where(keep, val, 0.0).astype(jnp.bfloat16)


def _conv(x, w, b, wd, th):
    bsz, hd, pw, c = x.shape
    cout = w.shape[-1]
    nb = hd // th
    zrow = jnp.zeros((bsz, 1, pw, c), jnp.bfloat16)
    if nb == 1:
        halo = jnp.zeros((bsz, 1, 2, pw, c), jnp.bfloat16)
    else:
        top = jnp.concatenate([zrow, x[:, th - 1:hd - 1:th]], axis=1)
        bot = jnp.concatenate([x[:, th::th], zrow], axis=1)
        halo = jnp.stack([top, bot], axis=2)
    wr = w.reshape(3, 3 * c, cout).astype(jnp.bfloat16)
    bf = b.reshape(1, cout)
    body = functools.partial(_conv_kernel, th=th, pw=pw, wd=wd, c=c)
    out = pl.pallas_call(
        body,
        out_shape=jax.ShapeDtypeStruct((bsz, hd, pw, cout), jnp.bfloat16),
        grid=(bsz * nb,),
        in_specs=[
            pl.BlockSpec((1, th, pw, c),
                         lambda g, nb=nb: (g // nb, g % nb, 0, 0)),
            pl.BlockSpec((1, 1, 2, pw, c),
                         lambda g, nb=nb: (g // nb, g % nb, 0, 0, 0)),
            pl.BlockSpec((3, 3 * c, cout), lambda g: (0, 0, 0)),
            pl.BlockSpec((1, cout), lambda g: (0, 0)),
        ],
        out_specs=pl.BlockSpec((1, th, pw, cout),
                               lambda g, nb=nb: (g // nb, g % nb, 0, 0)),
        compiler_params=pltpu.CompilerParams(
            dimension_semantics=("parallel",), vmem_limit_bytes=_VMEM),
    )(x, halo, wr, bf)
    return out


_L1_GROUPS = 32
_L1_LANES = 512


def _l1_kernel(x_ref, y_ref, o_ref):
    d = x_ref[...].astype(jnp.float32) - y_ref[...].astype(jnp.float32)
    o_ref[...] = jnp.sum(jnp.abs(d), axis=0)


def _l1_halves(f, wd):
    bsz, hd, pw, c = f.shape
    half = (bsz // 2) * hd * pw * c
    chunk = _L1_GROUPS * 8 * _L1_LANES
    nt = half // chunk
    assert nt * chunk == half, (f.shape, half, chunk)
    f3 = f.reshape(2 * nt * _L1_GROUPS, 8, _L1_LANES)
    partials = pl.pallas_call(
        _l1_kernel,
        out_shape=jax.ShapeDtypeStruct((nt * 8, _L1_LANES), jnp.float32),
        grid=(nt,),
        in_specs=[
            pl.BlockSpec((_L1_GROUPS, 8, _L1_LANES), lambda i: (i, 0, 0)),
            pl.BlockSpec((_L1_GROUPS, 8, _L1_LANES),
                         lambda i, nt=nt: (nt + i, 0, 0)),
        ],
        out_specs=pl.BlockSpec((8, _L1_LANES), lambda i: (i, 0)),
        compiler_params=pltpu.CompilerParams(
            dimension_semantics=("parallel",), vmem_limit_bytes=_VMEM),
    )(f3, f3)
    n_true = (bsz // 2) * hd * wd * c
    return jnp.sum(partials) / jnp.float32(n_true)


def _pool(f, wd):
    bsz, hd, pw, c = f.shape
    d = f[:, :, 1:wd + 1]
    p = d.reshape(bsz, hd // 2, 2, wd // 2, 2, c).max(axis=(2, 4))
    pw2 = _PW[wd // 2]
    return jnp.pad(p, ((0, 0), (0, 0), (1, pw2 - wd // 2 - 1), (0, 0)))


def kernel(inp, tgt,
           w0, b0, w1, b1, w2, b2, w3, b3, w4, b4,
           w5, b5, w6, b6, w7, b7, w8, b8, w9, b9):
    z = jnp.concatenate([inp, tgt], axis=0)
    rz = _resize_norm(z, 224)

    f = _conv1_1(rz, w0, b0)
    f = _conv(f, w1, b1, wd=224, th=56)
    loss = _l1_halves(f, 224)

    f = _pool(f, 224)
    f = _conv(f, w2, b2, wd=112, th=112)
    f = _conv(f, w3, b3, wd=112, th=112)
    loss = loss + _l1_halves(f, 112)

    f = _pool(f, 112)
    f = _conv(f, w4, b4, wd=56, th=56)
    f = _conv(f, w5, b5, wd=56, th=56)
    f = _conv(f, w6, b6, wd=56, th=56)
    loss = loss + _l1_halves(f, 56)

    f = _pool(f, 56)
    f = _conv(f, w7, b7, wd=28, th=28)
    f = _conv(f, w8, b8, wd=28, th=28)
    f = _conv(f, w9, b9, wd=28, th=28)
    loss = loss + _l1_halves(f, 28)
    return loss


# speedup vs baseline: 5.3768x; 1.2701x over previous
import functools

import numpy as np
import jax
import jax.numpy as jnp
from jax import lax
from jax.experimental import pallas as pl
from jax.experimental.pallas import tpu as pltpu

_MEAN = (0.485, 0.456, 0.406)
_STD = (0.229, 0.224, 0.225)

_VMEM = 60 * 1024 * 1024

_PW = {224: 232, 112: 120, 56: 64, 28: 32, 8: 16, 4: 8}


def _bilinear_matrix(out_size, in_size):
    mat = np.zeros((out_size, in_size), dtype=np.float32)
    scale = in_size / out_size
    for d in range(out_size):
        src = max((d + 0.5) * scale - 0.5, 0.0)
        i0 = min(int(np.floor(src)), in_size - 1)
        i1 = min(i0 + 1, in_size - 1)
        w1 = np.float32(src - i0)
        mat[d, i0] += np.float32(1.0) - w1
        mat[d, i1] += w1
    return jnp.asarray(mat)


def _resize_kernel(x_ref, rh_ref, rwt_ref, o_ref):
    c = lax.rem(pl.program_id(0), 3)
    t = jnp.dot(rh_ref[...], x_ref[0], preferred_element_type=jnp.float32)
    r = jnp.dot(t, rwt_ref[...], preferred_element_type=jnp.float32)
    inv_s = jnp.where(c == 0, np.float32(1.0 / _STD[0]),
                      jnp.where(c == 1, np.float32(1.0 / _STD[1]),
                                np.float32(1.0 / _STD[2])))
    m_s = jnp.where(c == 0, np.float32(_MEAN[0] / _STD[0]),
                    jnp.where(c == 1, np.float32(_MEAN[1] / _STD[1]),
                              np.float32(_MEAN[2] / _STD[2])))
    o_ref[0] = (r * inv_s - m_s).astype(jnp.bfloat16)


def _resize_norm(z, out_hw=224):
    b, c, h, w = z.shape
    rh = _bilinear_matrix(out_hw, h)
    rwt = _bilinear_matrix(out_hw, w).T
    zf = z.reshape(b * c, h, w)
    out = pl.pallas_call(
        _resize_kernel,
        out_shape=jax.ShapeDtypeStruct((b * c, out_hw, out_hw), jnp.bfloat16),
        grid=(b * c,),
        in_specs=[
            pl.BlockSpec((1, h, w), lambda i: (i, 0, 0)),
            pl.BlockSpec((out_hw, h), lambda i: (0, 0)),
            pl.BlockSpec((w, out_hw), lambda i: (0, 0)),
        ],
        out_specs=pl.BlockSpec((1, out_hw, out_hw), lambda i: (i, 0, 0)),
        compiler_params=pltpu.CompilerParams(
            dimension_semantics=("parallel",), vmem_limit_bytes=_VMEM),
    )(zf, rh, rwt)
    return out.reshape(b, c, out_hw, out_hw)


def _c11_kernel(a_ref, w_ref, b_ref, o_ref, *, th, wd, pw, cout):
    acc = lax.dot_general(a_ref[...], w_ref[...],
                          (((0,), (0,)), ((), ())),
                          preferred_element_type=jnp.float32)
    val = jnp.maximum(acc + b_ref[...], 0.0).astype(jnp.bfloat16)
    r3 = val.reshape(th, wd, cout)
    zr = jnp.zeros((th, pw - wd, cout), jnp.bfloat16)
    o_ref[0] = jnp.concatenate([r3, zr], axis=1)


def _conv1_1(rz, w, b, th=56):
    bsz, cin, hd, wd = rz.shape
    cout = w.shape[-1]
    pw = _PW[wd]
    xp = jnp.pad(rz, ((0, 0), (0, 0), (1, 1), (1, 1)))
    cols = [xp[:, c, dy:dy + hd, dx:dx + wd]
            for dy in range(3) for dx in range(3) for c in range(cin)]
    a_t = jnp.stack(cols, axis=0).reshape(9 * cin, bsz * hd * wd)
    wf = w.reshape(9 * cin, cout).astype(jnp.bfloat16)
    bf = b.reshape(1, cout)
    nb = hd // th
    body = functools.partial(_c11_kernel, th=th, wd=wd, pw=pw, cout=cout)
    out = pl.pallas_call(
        body,
        out_shape=jax.ShapeDtypeStruct((bsz, hd, pw, cout), jnp.bfloat16),
        grid=(bsz * nb,),
        in_specs=[
            pl.BlockSpec((9 * cin, th * wd), lambda g: (0, g)),
            pl.BlockSpec((9 * cin, cout), lambda g: (0, 0)),
            pl.BlockSpec((1, cout), lambda g: (0, 0)),
        ],
        out_specs=pl.BlockSpec((1, th, pw, cout),
                               lambda g, nb=nb: (g // nb, g % nb, 0, 0)),
        compiler_params=pltpu.CompilerParams(
            dimension_semantics=("parallel",), vmem_limit_bytes=_VMEM),
    )(a_t, wf, bf)
    return out


def _conv_kernel(xm_ref, xh_ref, w_ref, b_ref, o_ref, *, th, pw, wd, c):
    xm = xm_ref[0]
    top = xh_ref[0, 0, 0:1]
    bot = xh_ref[0, 0, 1:2]
    xf = jnp.concatenate([top, xm, bot], axis=0)
    z1 = jnp.zeros((th + 2, 1, c), jnp.bfloat16)
    left = jnp.concatenate([z1, xf[:, :pw - 1]], axis=1)
    right = jnp.concatenate([xf[:, 1:], z1], axis=1)
    a = jnp.concatenate([left, xf, right], axis=2)
    cout = o_ref.shape[-1]
    acc = jnp.zeros((th * pw, cout), jnp.float32)
    for dy in range(3):
        acc = acc + jnp.dot(a[dy:dy + th].reshape(th * pw, 3 * c), w_ref[dy],
                            preferred_element_type=jnp.float32)
    val = jnp.maximum(acc + b_ref[...], 0.0).reshape(th, pw, cout)
    ci = lax.broadcasted_iota(jnp.int32, (th, pw, cout), 1)
    o_ref[0] = jnp.where(ci < wd, val, 0.0).astype(jnp.bfloat16)


def _conv(x, w, b, wd, th):
    bsz, hd, pw, c = x.shape
    cout = w.shape[-1]
    nb = hd // th
    zrow = jnp.zeros((bsz, 1, pw, c), jnp.bfloat16)
    if nb == 1:
        halo = jnp.zeros((bsz, 1, 2, pw, c), jnp.bfloat16)
    else:
        top = jnp.concatenate([zrow, x[:, th - 1:hd - 1:th]], axis=1)
        bot = jnp.concatenate([x[:, th::th], zrow], axis=1)
        halo = jnp.stack([top, bot], axis=2)
    wr = w.reshape(3, 3 * c, cout).astype(jnp.bfloat16)
    bf = b.reshape(1, cout)
    body = functools.partial(_conv_kernel, th=th, pw=pw, wd=wd, c=c)
    out = pl.pallas_call(
        body,
        out_shape=jax.ShapeDtypeStruct((bsz, hd, pw, cout), jnp.bfloat16),
        grid=(bsz * nb,),
        in_specs=[
            pl.BlockSpec((1, th, pw, c),
                         lambda g, nb=nb: (g // nb, g % nb, 0, 0)),
            pl.BlockSpec((1, 1, 2, pw, c),
                         lambda g, nb=nb: (g // nb, g % nb, 0, 0, 0)),
            pl.BlockSpec((3, 3 * c, cout), lambda g: (0, 0, 0)),
            pl.BlockSpec((1, cout), lambda g: (0, 0)),
        ],
        out_specs=pl.BlockSpec((1, th, pw, cout),
                               lambda g, nb=nb: (g // nb, g % nb, 0, 0)),
        compiler_params=pltpu.CompilerParams(
            dimension_semantics=("parallel",), vmem_limit_bytes=_VMEM),
    )(x, halo, wr, bf)
    return out


def _l1_kernel(x_ref, y_ref, o_ref):
    d = x_ref[0].astype(jnp.float32) - y_ref[0].astype(jnp.float32)
    o_ref[0] = jnp.sum(jnp.abs(d), axis=(0, 1)).reshape(1, -1)


def _l1_halves(f, wd, thl):
    bsz, hd, pw, c = f.shape
    nbl = hd // thl
    g = (bsz // 2) * nbl
    partials = pl.pallas_call(
        _l1_kernel,
        out_shape=jax.ShapeDtypeStruct((g, 1, c), jnp.float32),
        grid=(g,),
        in_specs=[
            pl.BlockSpec((1, thl, pw, c),
                         lambda i, nbl=nbl: (i // nbl, i % nbl, 0, 0)),
            pl.BlockSpec((1, thl, pw, c),
                         lambda i, nbl=nbl, h=bsz // 2: (h + i // nbl,
                                                         i % nbl, 0, 0)),
        ],
        out_specs=pl.BlockSpec((1, 1, c), lambda i: (i, 0, 0)),
        compiler_params=pltpu.CompilerParams(
            dimension_semantics=("parallel",), vmem_limit_bytes=_VMEM),
    )(f, f)
    n_true = (bsz // 2) * hd * wd * c
    return jnp.sum(partials) / jnp.float32(n_true)


def _pool(f, wd):
    bsz, hd, pw, c = f.shape
    d = f[:, :, 0:wd]
    p = d.reshape(bsz, hd // 2, 2, wd // 2, 2, c).max(axis=(2, 4))
    pw2 = _PW[wd // 2]
    return jnp.pad(p, ((0, 0), (0, 0), (0, pw2 - wd // 2), (0, 0)))


def kernel(inp, tgt,
           w0, b0, w1, b1, w2, b2, w3, b3, w4, b4,
           w5, b5, w6, b6, w7, b7, w8, b8, w9, b9):
    z = jnp.concatenate([inp, tgt], axis=0)
    rz = _resize_norm(z, 224)

    f = _conv1_1(rz, w0, b0)
    f = _conv(f, w1, b1, wd=224, th=56)
    loss = _l1_halves(f, 224, thl=56)

    f = _pool(f, 224)
    f = _conv(f, w2, b2, wd=112, th=112)
    f = _conv(f, w3, b3, wd=112, th=112)
    loss = loss + _l1_halves(f, 112, thl=112)

    f = _pool(f, 112)
    f = _conv(f, w4, b4, wd=56, th=56)
    f = _conv(f, w5, b5, wd=56, th=56)
    f = _conv(f, w6, b6, wd=56, th=56)
    loss = loss + _l1_halves(f, 56, thl=56)

    f = _pool(f, 56)
    f = _conv(f, w7, b7, wd=28, th=28)
    f = _conv(f, w8, b8, wd=28, th=28)
    f = _conv(f, w9, b9, wd=28, th=28)
    loss = loss + _l1_halves(f, 28, thl=28)
    return loss


# speedup vs baseline: 5.5624x; 1.0345x over previous
import functools

import numpy as np
import jax
import jax.numpy as jnp
from jax import lax
from jax.experimental import pallas as pl
from jax.experimental.pallas import tpu as pltpu

_MEAN = (0.485, 0.456, 0.406)
_STD = (0.229, 0.224, 0.225)

_VMEM = 60 * 1024 * 1024

_PW = {224: 232, 112: 120, 56: 64, 28: 32, 8: 16, 4: 8}


def _bilinear_matrix(out_size, in_size):
    mat = np.zeros((out_size, in_size), dtype=np.float32)
    scale = in_size / out_size
    for d in range(out_size):
        src = max((d + 0.5) * scale - 0.5, 0.0)
        i0 = min(int(np.floor(src)), in_size - 1)
        i1 = min(i0 + 1, in_size - 1)
        w1 = np.float32(src - i0)
        mat[d, i0] += np.float32(1.0) - w1
        mat[d, i1] += w1
    return jnp.asarray(mat)


def _resize_kernel(x_ref, rh_ref, rwt_ref, o_ref):
    c = lax.rem(pl.program_id(0), 3)
    t = jnp.dot(rh_ref[...], x_ref[0], preferred_element_type=jnp.float32)
    r = jnp.dot(t, rwt_ref[...], preferred_element_type=jnp.float32)
    inv_s = jnp.where(c == 0, np.float32(1.0 / _STD[0]),
                      jnp.where(c == 1, np.float32(1.0 / _STD[1]),
                                np.float32(1.0 / _STD[2])))
    m_s = jnp.where(c == 0, np.float32(_MEAN[0] / _STD[0]),
                    jnp.where(c == 1, np.float32(_MEAN[1] / _STD[1]),
                              np.float32(_MEAN[2] / _STD[2])))
    o_ref[0] = (r * inv_s - m_s).astype(jnp.bfloat16)


def _resize_norm(z, out_hw=224):
    b, c, h, w = z.shape
    rh = _bilinear_matrix(out_hw, h)
    rwt = _bilinear_matrix(out_hw, w).T
    zf = z.reshape(b * c, h, w)
    out = pl.pallas_call(
        _resize_kernel,
        out_shape=jax.ShapeDtypeStruct((b * c, out_hw, out_hw), jnp.bfloat16),
        grid=(b * c,),
        in_specs=[
            pl.BlockSpec((1, h, w), lambda i: (i, 0, 0)),
            pl.BlockSpec((out_hw, h), lambda i: (0, 0)),
            pl.BlockSpec((w, out_hw), lambda i: (0, 0)),
        ],
        out_specs=pl.BlockSpec((1, out_hw, out_hw), lambda i: (i, 0, 0)),
        compiler_params=pltpu.CompilerParams(
            dimension_semantics=("parallel",), vmem_limit_bytes=_VMEM),
    )(zf, rh, rwt)
    return out.reshape(b, c, out_hw, out_hw)


def _c11_kernel(a_ref, w_ref, b_ref, o_ref, *, th, wd, pw, cout):
    acc = lax.dot_general(a_ref[...], w_ref[...],
                          (((0,), (0,)), ((), ())),
                          preferred_element_type=jnp.float32)
    val = jnp.maximum(acc + b_ref[...], 0.0).astype(jnp.bfloat16)
    r3 = val.reshape(th, wd, cout)
    zr = jnp.zeros((th, pw - wd, cout), jnp.bfloat16)
    o_ref[0] = jnp.concatenate([r3, zr], axis=1)


def _conv1_1(rz, w, b, th=56):
    bsz, cin, hd, wd = rz.shape
    cout = w.shape[-1]
    pw = _PW[wd]
    xp = jnp.pad(rz, ((0, 0), (0, 0), (1, 1), (1, 1)))
    cols = [xp[:, c, dy:dy + hd, dx:dx + wd]
            for dy in range(3) for dx in range(3) for c in range(cin)]
    a_t = jnp.stack(cols, axis=0).reshape(9 * cin, bsz * hd * wd)
    wf = w.reshape(9 * cin, cout).astype(jnp.bfloat16)
    bf = b.reshape(1, cout)
    nb = hd // th
    body = functools.partial(_c11_kernel, th=th, wd=wd, pw=pw, cout=cout)
    out = pl.pallas_call(
        body,
        out_shape=jax.ShapeDtypeStruct((bsz, hd, pw, cout), jnp.bfloat16),
        grid=(bsz * nb,),
        in_specs=[
            pl.BlockSpec((9 * cin, th * wd), lambda g: (0, g)),
            pl.BlockSpec((9 * cin, cout), lambda g: (0, 0)),
            pl.BlockSpec((1, cout), lambda g: (0, 0)),
        ],
        out_specs=pl.BlockSpec((1, th, pw, cout),
                               lambda g, nb=nb: (g // nb, g % nb, 0, 0)),
        compiler_params=pltpu.CompilerParams(
            dimension_semantics=("parallel",), vmem_limit_bytes=_VMEM),
    )(a_t, wf, bf)
    return out


def _conv_body(xm, top, bot, w_ref, b_ref, th, pw, wd, c):
    xf = jnp.concatenate([top, xm, bot], axis=0)
    z1 = jnp.zeros((th + 2, 1, c), jnp.bfloat16)
    left = jnp.concatenate([z1, xf[:, :pw - 1]], axis=1)
    right = jnp.concatenate([xf[:, 1:], z1], axis=1)
    a = jnp.concatenate([left, xf, right], axis=2)
    cout = w_ref.shape[-1]
    acc = jnp.zeros((th * pw, cout), jnp.float32)
    for dy in range(3):
        acc = acc + jnp.dot(a[dy:dy + th].reshape(th * pw, 3 * c), w_ref[dy],
                            preferred_element_type=jnp.float32)
    val = jnp.maximum(acc + b_ref[...], 0.0).reshape(th, pw, cout)
    ci = lax.broadcasted_iota(jnp.int32, (th, pw, cout), 1)
    return jnp.where(ci < wd, val, 0.0)


def _conv_kernel(xm_ref, xh_ref, w_ref, b_ref, o_ref, *, th, pw, wd, c):
    val = _conv_body(xm_ref[0], xh_ref[0, 0, 0:1], xh_ref[0, 0, 1:2],
                     w_ref, b_ref, th, pw, wd, c)
    o_ref[0] = val.astype(jnp.bfloat16)


def _conv(x, w, b, wd, th):
    bsz, hd, pw, c = x.shape
    cout = w.shape[-1]
    nb = hd // th
    zrow = jnp.zeros((bsz, 1, pw, c), jnp.bfloat16)
    if nb == 1:
        halo = jnp.zeros((bsz, 1, 2, pw, c), jnp.bfloat16)
    else:
        top = jnp.concatenate([zrow, x[:, th - 1:hd - 1:th]], axis=1)
        bot = jnp.concatenate([x[:, th::th], zrow], axis=1)
        halo = jnp.stack([top, bot], axis=2)
    wr = w.reshape(3, 3 * c, cout).astype(jnp.bfloat16)
    bf = b.reshape(1, cout)
    body = functools.partial(_conv_kernel, th=th, pw=pw, wd=wd, c=c)
    out = pl.pallas_call(
        body,
        out_shape=jax.ShapeDtypeStruct((bsz, hd, pw, cout), jnp.bfloat16),
        grid=(bsz * nb,),
        in_specs=[
            pl.BlockSpec((1, th, pw, c),
                         lambda g, nb=nb: (g // nb, g % nb, 0, 0)),
            pl.BlockSpec((1, 1, 2, pw, c),
                         lambda g, nb=nb: (g // nb, g % nb, 0, 0, 0)),
            pl.BlockSpec((3, 3 * c, cout), lambda g: (0, 0, 0)),
            pl.BlockSpec((1, cout), lambda g: (0, 0)),
        ],
        out_specs=pl.BlockSpec((1, th, pw, cout),
                               lambda g, nb=nb: (g // nb, g % nb, 0, 0)),
        compiler_params=pltpu.CompilerParams(
            dimension_semantics=("parallel",), vmem_limit_bytes=_VMEM),
    )(x, halo, wr, bf)
    return out


def _conv_pair_kernel(xm_ref, xh_ref, w_ref, b_ref, o_ref, p_ref,
                      *, th, pw, wd, c):
    v0 = _conv_body(xm_ref[0, 0], xh_ref[0, 0, 0, 0:1], xh_ref[0, 0, 0, 1:2],
                    w_ref, b_ref, th, pw, wd, c)
    v1 = _conv_body(xm_ref[1, 0], xh_ref[1, 0, 0, 0:1], xh_ref[1, 0, 0, 1:2],
                    w_ref, b_ref, th, pw, wd, c)
    o_ref[0, 0] = v0.astype(jnp.bfloat16)
    o_ref[1, 0] = v1.astype(jnp.bfloat16)
    p_ref[0] = jnp.sum(jnp.abs(v0 - v1), axis=(0, 1)).reshape(1, -1)


def _conv_pair_l1(x, w, b, wd, th):
    bsz, hd, pw, c = x.shape
    half = bsz // 2
    cout = w.shape[-1]
    nb = hd // th
    zrow = jnp.zeros((bsz, 1, pw, c), jnp.bfloat16)
    if nb == 1:
        halo = jnp.zeros((bsz, 1, 2, pw, c), jnp.bfloat16)
    else:
        top = jnp.concatenate([zrow, x[:, th - 1:hd - 1:th]], axis=1)
        bot = jnp.concatenate([x[:, th::th], zrow], axis=1)
        halo = jnp.stack([top, bot], axis=2)
    x5 = x.reshape(2, half, hd, pw, c)
    halo6 = halo.reshape(2, half, nb, 2, pw, c)
    wr = w.reshape(3, 3 * c, cout).astype(jnp.bfloat16)
    bf = b.reshape(1, cout)
    g = half * nb
    body = functools.partial(_conv_pair_kernel, th=th, pw=pw, wd=wd, c=c)
    out, partials = pl.pallas_call(
        body,
        out_shape=(
            jax.ShapeDtypeStruct((2, half, hd, pw, cout), jnp.bfloat16),
            jax.ShapeDtypeStruct((g, 1, cout), jnp.float32),
        ),
        grid=(g,),
        in_specs=[
            pl.BlockSpec((2, 1, th, pw, c),
                         lambda i, nb=nb: (0, i // nb, i % nb, 0, 0)),
            pl.BlockSpec((2, 1, 1, 2, pw, c),
                         lambda i, nb=nb: (0, i // nb, i % nb, 0, 0, 0)),
            pl.BlockSpec((3, 3 * c, cout), lambda i: (0, 0, 0)),
            pl.BlockSpec((1, cout), lambda i: (0, 0)),
        ],
        out_specs=(
            pl.BlockSpec((2, 1, th, pw, cout),
                         lambda i, nb=nb: (0, i // nb, i % nb, 0, 0)),
            pl.BlockSpec((1, 1, cout), lambda i: (i, 0, 0)),
        ),
        compiler_params=pltpu.CompilerParams(
            dimension_semantics=("parallel",), vmem_limit_bytes=_VMEM),
    )(x5, halo6, wr, bf)
    n_true = half * hd * wd * cout
    return out.reshape(bsz, hd, pw, cout), jnp.sum(partials) / jnp.float32(n_true)


def _l1_kernel(x_ref, y_ref, o_ref):
    d = x_ref[0].astype(jnp.float32) - y_ref[0].astype(jnp.float32)
    o_ref[0] = jnp.sum(jnp.abs(d), axis=(0, 1)).reshape(1, -1)


def _l1_halves(f, wd, thl):
    bsz, hd, pw, c = f.shape
    nbl = hd // thl
    g = (bsz // 2) * nbl
    partials = pl.pallas_call(
        _l1_kernel,
        out_shape=jax.ShapeDtypeStruct((g, 1, c), jnp.float32),
        grid=(g,),
        in_specs=[
            pl.BlockSpec((1, thl, pw, c),
                         lambda i, nbl=nbl: (i // nbl, i % nbl, 0, 0)),
            pl.BlockSpec((1, thl, pw, c),
                         lambda i, nbl=nbl, h=bsz // 2: (h + i // nbl,
                                                         i % nbl, 0, 0)),
        ],
        out_specs=pl.BlockSpec((1, 1, c), lambda i: (i, 0, 0)),
        compiler_params=pltpu.CompilerParams(
            dimension_semantics=("parallel",), vmem_limit_bytes=_VMEM),
    )(f, f)
    n_true = (bsz // 2) * hd * wd * c
    return jnp.sum(partials) / jnp.float32(n_true)


def _pool(f, wd):
    bsz, hd, pw, c = f.shape
    d = f[:, :, 0:wd]
    p = d.reshape(bsz, hd // 2, 2, wd // 2, 2, c).max(axis=(2, 4))
    pw2 = _PW[wd // 2]
    return jnp.pad(p, ((0, 0), (0, 0), (0, pw2 - wd // 2), (0, 0)))


def kernel(inp, tgt,
           w0, b0, w1, b1, w2, b2, w3, b3, w4, b4,
           w5, b5, w6, b6, w7, b7, w8, b8, w9, b9):
    z = jnp.concatenate([inp, tgt], axis=0)
    rz = _resize_norm(z, 224)

    f = _conv1_1(rz, w0, b0)
    f, l1 = _conv_pair_l1(f, w1, b1, wd=224, th=56)
    loss = l1

    f = _pool(f, 224)
    f = _conv(f, w2, b2, wd=112, th=112)
    f, l1 = _conv_pair_l1(f, w3, b3, wd=112, th=56)
    loss = loss + l1

    f = _pool(f, 112)
    f = _conv(f, w4, b4, wd=56, th=56)
    f = _conv(f, w5, b5, wd=56, th=56)
    f, l1 = _conv_pair_l1(f, w6, b6, wd=56, th=56)
    loss = loss + l1

    f = _pool(f, 56)
    f = _conv(f, w7, b7, wd=28, th=28)
    f = _conv(f, w8, b8, wd=28, th=28)
    f, l1 = _conv_pair_l1(f, w9, b9, wd=28, th=28)
    loss = loss + l1
    return loss


# speedup vs baseline: 7.1993x; 1.2943x over previous
import functools

import numpy as np
import jax
import jax.numpy as jnp
from jax import lax
from jax.experimental import pallas as pl
from jax.experimental.pallas import tpu as pltpu

_MEAN = (0.485, 0.456, 0.406)
_STD = (0.229, 0.224, 0.225)

_VMEM = 60 * 1024 * 1024

_PW = {224: 232, 112: 120, 56: 64, 28: 32, 8: 16, 4: 8}


def _bilinear_matrix(out_size, in_size):
    mat = np.zeros((out_size, in_size), dtype=np.float32)
    scale = in_size / out_size
    for d in range(out_size):
        src = max((d + 0.5) * scale - 0.5, 0.0)
        i0 = min(int(np.floor(src)), in_size - 1)
        i1 = min(i0 + 1, in_size - 1)
        w1 = np.float32(src - i0)
        mat[d, i0] += np.float32(1.0) - w1
        mat[d, i1] += w1
    return jnp.asarray(mat)


def _resize_kernel(x_ref, rh_ref, rwt_ref, o_ref):
    c = lax.rem(pl.program_id(0), 3)
    t = jnp.dot(rh_ref[...], x_ref[0], preferred_element_type=jnp.float32)
    r = jnp.dot(t, rwt_ref[...], preferred_element_type=jnp.float32)
    inv_s = jnp.where(c == 0, np.float32(1.0 / _STD[0]),
                      jnp.where(c == 1, np.float32(1.0 / _STD[1]),
                                np.float32(1.0 / _STD[2])))
    m_s = jnp.where(c == 0, np.float32(_MEAN[0] / _STD[0]),
                    jnp.where(c == 1, np.float32(_MEAN[1] / _STD[1]),
                              np.float32(_MEAN[2] / _STD[2])))
    o_ref[0] = (r * inv_s - m_s).astype(jnp.bfloat16)


def _resize_norm(z, out_hw=224):
    b, c, h, w = z.shape
    rh = _bilinear_matrix(out_hw, h)
    rwt = _bilinear_matrix(out_hw, w).T
    zf = z.reshape(b * c, h, w)
    out = pl.pallas_call(
        _resize_kernel,
        out_shape=jax.ShapeDtypeStruct((b * c, out_hw, out_hw), jnp.bfloat16),
        grid=(b * c,),
        in_specs=[
            pl.BlockSpec((1, h, w), lambda i: (i, 0, 0)),
            pl.BlockSpec((out_hw, h), lambda i: (0, 0)),
            pl.BlockSpec((w, out_hw), lambda i: (0, 0)),
        ],
        out_specs=pl.BlockSpec((1, out_hw, out_hw), lambda i: (i, 0, 0)),
        compiler_params=pltpu.CompilerParams(
            dimension_semantics=("parallel",), vmem_limit_bytes=_VMEM),
    )(zf, rh, rwt)
    return out.reshape(b, c, out_hw, out_hw)


def _c11_kernel(a_ref, w_ref, b_ref, o_ref, *, th, wd, pw, cout):
    acc = lax.dot_general(a_ref[...], w_ref[...],
                          (((0,), (0,)), ((), ())),
                          preferred_element_type=jnp.float32)
    val = jnp.maximum(acc + b_ref[...], 0.0).astype(jnp.bfloat16)
    r3 = val.reshape(th, wd, cout)
    zr = jnp.zeros((th, pw - wd, cout), jnp.bfloat16)
    o_ref[0] = jnp.concatenate([r3, zr], axis=1)


def _conv1_1(rz, w, b, th=56):
    bsz, cin, hd, wd = rz.shape
    cout = w.shape[-1]
    pw = _PW[wd]
    xp = jnp.pad(rz, ((0, 0), (0, 0), (1, 1), (1, 1)))
    cols = [xp[:, c, dy:dy + hd, dx:dx + wd]
            for dy in range(3) for dx in range(3) for c in range(cin)]
    a_t = jnp.stack(cols, axis=0).reshape(9 * cin, bsz * hd * wd)
    wf = w.reshape(9 * cin, cout).astype(jnp.bfloat16)
    bf = b.reshape(1, cout)
    nb = hd // th
    body = functools.partial(_c11_kernel, th=th, wd=wd, pw=pw, cout=cout)
    out = pl.pallas_call(
        body,
        out_shape=jax.ShapeDtypeStruct((bsz, hd, pw, cout), jnp.bfloat16),
        grid=(bsz * nb,),
        in_specs=[
            pl.BlockSpec((9 * cin, th * wd), lambda g: (0, g)),
            pl.BlockSpec((9 * cin, cout), lambda g: (0, 0)),
            pl.BlockSpec((1, cout), lambda g: (0, 0)),
        ],
        out_specs=pl.BlockSpec((1, th, pw, cout),
                               lambda g, nb=nb: (g // nb, g % nb, 0, 0)),
        compiler_params=pltpu.CompilerParams(
            dimension_semantics=("parallel",), vmem_limit_bytes=_VMEM),
    )(a_t, wf, bf)
    return out


def _conv_body(xm, top, bot, w_ref, b_ref, th, pw, wd, c):
    xf = jnp.concatenate([top, xm, bot], axis=0)
    z1 = jnp.zeros((th + 2, 1, c), jnp.bfloat16)
    left = jnp.concatenate([z1, xf[:, :pw - 1]], axis=1)
    right = jnp.concatenate([xf[:, 1:], z1], axis=1)
    a = jnp.concatenate([left, xf, right], axis=2)
    cout = w_ref.shape[-1]
    acc = jnp.zeros((th * pw, cout), jnp.float32)
    for dy in range(3):
        acc = acc + jnp.dot(a[dy:dy + th].reshape(th * pw, 3 * c), w_ref[dy],
                            preferred_element_type=jnp.float32)
    val = jnp.maximum(acc + b_ref[...], 0.0).reshape(th, pw, cout)
    ci = lax.broadcasted_iota(jnp.int32, (th, pw, cout), 1)
    return jnp.where(ci < wd, val, 0.0)


def _conv_kernel(xm_ref, xh_ref, w_ref, b_ref, o_ref, *, th, pw, wd, c):
    val = _conv_body(xm_ref[0], xh_ref[0, 0, 0:1], xh_ref[0, 0, 1:2],
                     w_ref, b_ref, th, pw, wd, c)
    o_ref[0] = val.astype(jnp.bfloat16)


def _conv(x, w, b, wd, th):
    bsz, hd, pw, c = x.shape
    cout = w.shape[-1]
    nb = hd // th
    zrow = jnp.zeros((bsz, 1, pw, c), jnp.bfloat16)
    if nb == 1:
        halo = jnp.zeros((bsz, 1, 2, pw, c), jnp.bfloat16)
    else:
        top = jnp.concatenate([zrow, x[:, th - 1:hd - 1:th]], axis=1)
        bot = jnp.concatenate([x[:, th::th], zrow], axis=1)
        halo = jnp.stack([top, bot], axis=2)
    wr = w.reshape(3, 3 * c, cout).astype(jnp.bfloat16)
    bf = b.reshape(1, cout)
    body = functools.partial(_conv_kernel, th=th, pw=pw, wd=wd, c=c)
    out = pl.pallas_call(
        body,
        out_shape=jax.ShapeDtypeStruct((bsz, hd, pw, cout), jnp.bfloat16),
        grid=(bsz * nb,),
        in_specs=[
            pl.BlockSpec((1, th, pw, c),
                         lambda g, nb=nb: (g // nb, g % nb, 0, 0)),
            pl.BlockSpec((1, 1, 2, pw, c),
                         lambda g, nb=nb: (g // nb, g % nb, 0, 0, 0)),
            pl.BlockSpec((3, 3 * c, cout), lambda g: (0, 0, 0)),
            pl.BlockSpec((1, cout), lambda g: (0, 0)),
        ],
        out_specs=pl.BlockSpec((1, th, pw, cout),
                               lambda g, nb=nb: (g // nb, g % nb, 0, 0)),
        compiler_params=pltpu.CompilerParams(
            dimension_semantics=("parallel",), vmem_limit_bytes=_VMEM),
    )(x, halo, wr, bf)
    return out


def _conv_pair_kernel(xm_ref, xh_ref, w_ref, b_ref, o_ref, p_ref,
                      *, th, pw, wd, c):
    v0 = _conv_body(xm_ref[0, 0], xh_ref[0, 0, 0, 0:1], xh_ref[0, 0, 0, 1:2],
                    w_ref, b_ref, th, pw, wd, c)
    v1 = _conv_body(xm_ref[1, 0], xh_ref[1, 0, 0, 0:1], xh_ref[1, 0, 0, 1:2],
                    w_ref, b_ref, th, pw, wd, c)
    o_ref[0, 0] = v0.astype(jnp.bfloat16)
    o_ref[1, 0] = v1.astype(jnp.bfloat16)
    p_ref[0] = jnp.sum(jnp.abs(v0 - v1), axis=(0, 1)).reshape(1, -1)


def _conv_pair_l1(x, w, b, wd, th):
    bsz, hd, pw, c = x.shape
    half = bsz // 2
    cout = w.shape[-1]
    nb = hd // th
    zrow = jnp.zeros((bsz, 1, pw, c), jnp.bfloat16)
    if nb == 1:
        halo = jnp.zeros((bsz, 1, 2, pw, c), jnp.bfloat16)
    else:
        top = jnp.concatenate([zrow, x[:, th - 1:hd - 1:th]], axis=1)
        bot = jnp.concatenate([x[:, th::th], zrow], axis=1)
        halo = jnp.stack([top, bot], axis=2)
    x5 = x.reshape(2, half, hd, pw, c)
    halo6 = halo.reshape(2, half, nb, 2, pw, c)
    wr = w.reshape(3, 3 * c, cout).astype(jnp.bfloat16)
    bf = b.reshape(1, cout)
    g = half * nb
    body = functools.partial(_conv_pair_kernel, th=th, pw=pw, wd=wd, c=c)
    out, partials = pl.pallas_call(
        body,
        out_shape=(
            jax.ShapeDtypeStruct((2, half, hd, pw, cout), jnp.bfloat16),
            jax.ShapeDtypeStruct((g, 1, cout), jnp.float32),
        ),
        grid=(g,),
        in_specs=[
            pl.BlockSpec((2, 1, th, pw, c),
                         lambda i, nb=nb: (0, i // nb, i % nb, 0, 0)),
            pl.BlockSpec((2, 1, 1, 2, pw, c),
                         lambda i, nb=nb: (0, i // nb, i % nb, 0, 0, 0)),
            pl.BlockSpec((3, 3 * c, cout), lambda i: (0, 0, 0)),
            pl.BlockSpec((1, cout), lambda i: (0, 0)),
        ],
        out_specs=(
            pl.BlockSpec((2, 1, th, pw, cout),
                         lambda i, nb=nb: (0, i // nb, i % nb, 0, 0)),
            pl.BlockSpec((1, 1, cout), lambda i: (i, 0, 0)),
        ),
        compiler_params=pltpu.CompilerParams(
            dimension_semantics=("parallel",), vmem_limit_bytes=_VMEM),
    )(x5, halo6, wr, bf)
    n_true = half * hd * wd * cout
    return out.reshape(bsz, hd, pw, cout), jnp.sum(partials) / jnp.float32(n_true)


def _l1_kernel(x_ref, y_ref, o_ref):
    d = x_ref[0].astype(jnp.float32) - y_ref[0].astype(jnp.float32)
    o_ref[0] = jnp.sum(jnp.abs(d), axis=(0, 1)).reshape(1, -1)


def _l1_halves(f, wd, thl):
    bsz, hd, pw, c = f.shape
    nbl = hd // thl
    g = (bsz // 2) * nbl
    partials = pl.pallas_call(
        _l1_kernel,
        out_shape=jax.ShapeDtypeStruct((g, 1, c), jnp.float32),
        grid=(g,),
        in_specs=[
            pl.BlockSpec((1, thl, pw, c),
                         lambda i, nbl=nbl: (i // nbl, i % nbl, 0, 0)),
            pl.BlockSpec((1, thl, pw, c),
                         lambda i, nbl=nbl, h=bsz // 2: (h + i // nbl,
                                                         i % nbl, 0, 0)),
        ],
        out_specs=pl.BlockSpec((1, 1, c), lambda i: (i, 0, 0)),
        compiler_params=pltpu.CompilerParams(
            dimension_semantics=("parallel",), vmem_limit_bytes=_VMEM),
    )(f, f)
    n_true = (bsz // 2) * hd * wd * c
    return jnp.sum(partials) / jnp.float32(n_true)


def _pool_kernel(x_ref, o_ref, *, thp, pw, pw2, c):
    v = x_ref[0]
    rp = jnp.max(v.reshape(thp, 2, pw, c), axis=1)
    sh = jnp.concatenate([rp[:, 1:], jnp.zeros((thp, 1, c), rp.dtype)], axis=1)
    cm = jnp.maximum(rp, sh)
    ds = cm.reshape(thp, pw // 2, 2, c)[:, :, 0]
    if pw2 > pw // 2:
        zr = jnp.zeros((thp, pw2 - pw // 2, c), rp.dtype)
        ds = jnp.concatenate([ds, zr], axis=1)
    o_ref[0] = ds


def _pool(f, wd, thp=None):
    bsz, hd, pw, c = f.shape
    pw2 = _PW[wd // 2]
    if thp is None:
        thp = hd // 2
    nbp = (hd // 2) // thp
    body = functools.partial(_pool_kernel, thp=thp, pw=pw, pw2=pw2, c=c)
    return pl.pallas_call(
        body,
        out_shape=jax.ShapeDtypeStruct((bsz, hd // 2, pw2, c), jnp.bfloat16),
        grid=(bsz * nbp,),
        in_specs=[
            pl.BlockSpec((1, 2 * thp, pw, c),
                         lambda g, nbp=nbp: (g // nbp, g % nbp, 0, 0)),
        ],
        out_specs=pl.BlockSpec((1, thp, pw2, c),
                               lambda g, nbp=nbp: (g // nbp, g % nbp, 0, 0)),
        compiler_params=pltpu.CompilerParams(
            dimension_semantics=("parallel",), vmem_limit_bytes=_VMEM),
    )(f)


def kernel(inp, tgt,
           w0, b0, w1, b1, w2, b2, w3, b3, w4, b4,
           w5, b5, w6, b6, w7, b7, w8, b8, w9, b9):
    z = jnp.concatenate([inp, tgt], axis=0)
    rz = _resize_norm(z, 224)

    f = _conv1_1(rz, w0, b0)
    f, l1 = _conv_pair_l1(f, w1, b1, wd=224, th=56)
    loss = l1

    f = _pool(f, 224, thp=56)
    f = _conv(f, w2, b2, wd=112, th=112)
    f, l1 = _conv_pair_l1(f, w3, b3, wd=112, th=56)
    loss = loss + l1

    f = _pool(f, 112)
    f = _conv(f, w4, b4, wd=56, th=56)
    f = _conv(f, w5, b5, wd=56, th=56)
    f, l1 = _conv_pair_l1(f, w6, b6, wd=56, th=56)
    loss = loss + l1

    f = _pool(f, 56)
    f = _conv(f, w7, b7, wd=28, th=28)
    f = _conv(f, w8, b8, wd=28, th=28)
    f, l1 = _conv_pair_l1(f, w9, b9, wd=28, th=28)
    loss = loss + l1
    return loss


# speedup vs baseline: 7.2759x; 1.0106x over previous
import functools

import numpy as np
import jax
import jax.numpy as jnp
from jax import lax
from jax.experimental import pallas as pl
from jax.experimental.pallas import tpu as pltpu

_MEAN = (0.485, 0.456, 0.406)
_STD = (0.229, 0.224, 0.225)

_VMEM = 60 * 1024 * 1024

_PW = {224: 232, 112: 120, 56: 64, 28: 32, 8: 16, 4: 8}


def _bilinear_matrix(out_size, in_size):
    mat = np.zeros((out_size, in_size), dtype=np.float32)
    scale = in_size / out_size
    for d in range(out_size):
        src = max((d + 0.5) * scale - 0.5, 0.0)
        i0 = min(int(np.floor(src)), in_size - 1)
        i1 = min(i0 + 1, in_size - 1)
        w1 = np.float32(src - i0)
        mat[d, i0] += np.float32(1.0) - w1
        mat[d, i1] += w1
    return jnp.asarray(mat)


def _resize_kernel(xi_ref, xt_ref, rh_ref, rwt_ref, o_ref):
    c = lax.rem(pl.program_id(0), 3)
    inv_s = jnp.where(c == 0, np.float32(1.0 / _STD[0]),
                      jnp.where(c == 1, np.float32(1.0 / _STD[1]),
                                np.float32(1.0 / _STD[2])))
    m_s = jnp.where(c == 0, np.float32(_MEAN[0] / _STD[0]),
                    jnp.where(c == 1, np.float32(_MEAN[1] / _STD[1]),
                              np.float32(_MEAN[2] / _STD[2])))
    for s, ref in enumerate((xi_ref, xt_ref)):
        t = jnp.dot(rh_ref[...], ref[0, 0], preferred_element_type=jnp.float32)
        r = jnp.dot(t, rwt_ref[...], preferred_element_type=jnp.float32)
        o_ref[s, 0, 0] = (r * inv_s - m_s).astype(jnp.bfloat16)


def _resize_norm(inp, tgt, out_hw=224):
    n, c3, h, w = inp.shape
    rh = _bilinear_matrix(out_hw, h)
    rwt = _bilinear_matrix(out_hw, w).T
    out = pl.pallas_call(
        _resize_kernel,
        out_shape=jax.ShapeDtypeStruct((2, n, c3, out_hw, out_hw),
                                       jnp.bfloat16),
        grid=(n * c3,),
        in_specs=[
            pl.BlockSpec((1, 1, h, w), lambda g: (g // 3, g % 3, 0, 0)),
            pl.BlockSpec((1, 1, h, w), lambda g: (g // 3, g % 3, 0, 0)),
            pl.BlockSpec((out_hw, h), lambda g: (0, 0)),
            pl.BlockSpec((w, out_hw), lambda g: (0, 0)),
        ],
        out_specs=pl.BlockSpec((2, 1, 1, out_hw, out_hw),
                               lambda g: (0, g // 3, g % 3, 0, 0)),
        compiler_params=pltpu.CompilerParams(
            dimension_semantics=("parallel",), vmem_limit_bytes=_VMEM),
    )(inp, tgt, rh, rwt)
    return out.reshape(2 * n, c3, out_hw, out_hw)


def _c11_kernel(a_ref, w_ref, b_ref, o_ref, *, th, wd, pw, cout):
    acc = lax.dot_general(a_ref[...], w_ref[...],
                          (((0,), (0,)), ((), ())),
                          preferred_element_type=jnp.float32)
    val = jnp.maximum(acc + b_ref[...], 0.0).astype(jnp.bfloat16)
    r3 = val.reshape(th, wd, cout)
    zr = jnp.zeros((th, pw - wd, cout), jnp.bfloat16)
    o_ref[0] = jnp.concatenate([r3, zr], axis=1)


def _conv1_1(rz, w, b, th=56):
    bsz, cin, hd, wd = rz.shape
    cout = w.shape[-1]
    pw = _PW[wd]
    xp = jnp.pad(rz, ((0, 0), (0, 0), (1, 1), (1, 1)))
    cols = [xp[:, c, dy:dy + hd, dx:dx + wd]
            for dy in range(3) for dx in range(3) for c in range(cin)]
    a_t = jnp.stack(cols, axis=0).reshape(9 * cin, bsz * hd * wd)
    wf = w.reshape(9 * cin, cout).astype(jnp.bfloat16)
    bf = b.reshape(1, cout)
    nb = hd // th
    body = functools.partial(_c11_kernel, th=th, wd=wd, pw=pw, cout=cout)
    out = pl.pallas_call(
        body,
        out_shape=jax.ShapeDtypeStruct((bsz, hd, pw, cout), jnp.bfloat16),
        grid=(bsz * nb,),
        in_specs=[
            pl.BlockSpec((9 * cin, th * wd), lambda g: (0, g)),
            pl.BlockSpec((9 * cin, cout), lambda g: (0, 0)),
            pl.BlockSpec((1, cout), lambda g: (0, 0)),
        ],
        out_specs=pl.BlockSpec((1, th, pw, cout),
                               lambda g, nb=nb: (g // nb, g % nb, 0, 0)),
        compiler_params=pltpu.CompilerParams(
            dimension_semantics=("parallel",), vmem_limit_bytes=_VMEM),
    )(a_t, wf, bf)
    return out


def _conv_body(xm, top, bot, w_ref, b_ref, th, pw, wd, c):
    xf = jnp.concatenate([top, xm, bot], axis=0)
    z1 = jnp.zeros((th + 2, 1, c), jnp.bfloat16)
    left = jnp.concatenate([z1, xf[:, :pw - 1]], axis=1)
    right = jnp.concatenate([xf[:, 1:], z1], axis=1)
    a = jnp.concatenate([left, xf, right], axis=2)
    cout = w_ref.shape[-1]
    acc = jnp.zeros((th * pw, cout), jnp.float32)
    for dy in range(3):
        acc = acc + jnp.dot(a[dy:dy + th].reshape(th * pw, 3 * c), w_ref[dy],
                            preferred_element_type=jnp.float32)
    val = jnp.maximum(acc + b_ref[...], 0.0).reshape(th, pw, cout)
    ci = lax.broadcasted_iota(jnp.int32, (th, pw, cout), 1)
    return jnp.where(ci < wd, val, 0.0)


def _conv_kernel(xm_ref, xh_ref, w_ref, b_ref, o_ref, *, th, pw, wd, c):
    val = _conv_body(xm_ref[0], xh_ref[0, 0, 0:1], xh_ref[0, 0, 1:2],
                     w_ref, b_ref, th, pw, wd, c)
    o_ref[0] = val.astype(jnp.bfloat16)


def _conv(x, w, b, wd, th):
    bsz, hd, pw, c = x.shape
    cout = w.shape[-1]
    nb = hd // th
    zrow = jnp.zeros((bsz, 1, pw, c), jnp.bfloat16)
    if nb == 1:
        halo = jnp.zeros((bsz, 1, 2, pw, c), jnp.bfloat16)
    else:
        top = jnp.concatenate([zrow, x[:, th - 1:hd - 1:th]], axis=1)
        bot = jnp.concatenate([x[:, th::th], zrow], axis=1)
        halo = jnp.stack([top, bot], axis=2)
    wr = w.reshape(3, 3 * c, cout).astype(jnp.bfloat16)
    bf = b.reshape(1, cout)
    body = functools.partial(_conv_kernel, th=th, pw=pw, wd=wd, c=c)
    out = pl.pallas_call(
        body,
        out_shape=jax.ShapeDtypeStruct((bsz, hd, pw, cout), jnp.bfloat16),
        grid=(bsz * nb,),
        in_specs=[
            pl.BlockSpec((1, th, pw, c),
                         lambda g, nb=nb: (g // nb, g % nb, 0, 0)),
            pl.BlockSpec((1, 1, 2, pw, c),
                         lambda g, nb=nb: (g // nb, g % nb, 0, 0, 0)),
            pl.BlockSpec((3, 3 * c, cout), lambda g: (0, 0, 0)),
            pl.BlockSpec((1, cout), lambda g: (0, 0)),
        ],
        out_specs=pl.BlockSpec((1, th, pw, cout),
                               lambda g, nb=nb: (g // nb, g % nb, 0, 0)),
        compiler_params=pltpu.CompilerParams(
            dimension_semantics=("parallel",), vmem_limit_bytes=_VMEM),
    )(x, halo, wr, bf)
    return out


def _conv_pair_kernel(xm_ref, xh_ref, w_ref, b_ref, o_ref, p_ref,
                      *, th, pw, wd, c):
    v0 = _conv_body(xm_ref[0, 0], xh_ref[0, 0, 0, 0:1], xh_ref[0, 0, 0, 1:2],
                    w_ref, b_ref, th, pw, wd, c)
    v1 = _conv_body(xm_ref[1, 0], xh_ref[1, 0, 0, 0:1], xh_ref[1, 0, 0, 1:2],
                    w_ref, b_ref, th, pw, wd, c)
    o_ref[0, 0] = v0.astype(jnp.bfloat16)
    o_ref[1, 0] = v1.astype(jnp.bfloat16)
    p_ref[0] = jnp.sum(jnp.abs(v0 - v1), axis=(0, 1)).reshape(1, -1)


def _conv_pair_l1(x, w, b, wd, th):
    bsz, hd, pw, c = x.shape
    half = bsz // 2
    cout = w.shape[-1]
    nb = hd // th
    zrow = jnp.zeros((bsz, 1, pw, c), jnp.bfloat16)
    if nb == 1:
        halo = jnp.zeros((bsz, 1, 2, pw, c), jnp.bfloat16)
    else:
        top = jnp.concatenate([zrow, x[:, th - 1:hd - 1:th]], axis=1)
        bot = jnp.concatenate([x[:, th::th], zrow], axis=1)
        halo = jnp.stack([top, bot], axis=2)
    x5 = x.reshape(2, half, hd, pw, c)
    halo6 = halo.reshape(2, half, nb, 2, pw, c)
    wr = w.reshape(3, 3 * c, cout).astype(jnp.bfloat16)
    bf = b.reshape(1, cout)
    g = half * nb
    body = functools.partial(_conv_pair_kernel, th=th, pw=pw, wd=wd, c=c)
    out, partials = pl.pallas_call(
        body,
        out_shape=(
            jax.ShapeDtypeStruct((2, half, hd, pw, cout), jnp.bfloat16),
            jax.ShapeDtypeStruct((g, 1, cout), jnp.float32),
        ),
        grid=(g,),
        in_specs=[
            pl.BlockSpec((2, 1, th, pw, c),
                         lambda i, nb=nb: (0, i // nb, i % nb, 0, 0)),
            pl.BlockSpec((2, 1, 1, 2, pw, c),
                         lambda i, nb=nb: (0, i // nb, i % nb, 0, 0, 0)),
            pl.BlockSpec((3, 3 * c, cout), lambda i: (0, 0, 0)),
            pl.BlockSpec((1, cout), lambda i: (0, 0)),
        ],
        out_specs=(
            pl.BlockSpec((2, 1, th, pw, cout),
                         lambda i, nb=nb: (0, i // nb, i % nb, 0, 0)),
            pl.BlockSpec((1, 1, cout), lambda i: (i, 0, 0)),
        ),
        compiler_params=pltpu.CompilerParams(
            dimension_semantics=("parallel",), vmem_limit_bytes=_VMEM),
    )(x5, halo6, wr, bf)
    n_true = half * hd * wd * cout
    return out.reshape(bsz, hd, pw, cout), jnp.sum(partials) / jnp.float32(n_true)


def _l1_kernel(x_ref, y_ref, o_ref):
    d = x_ref[0].astype(jnp.float32) - y_ref[0].astype(jnp.float32)
    o_ref[0] = jnp.sum(jnp.abs(d), axis=(0, 1)).reshape(1, -1)


def _l1_halves(f, wd, thl):
    bsz, hd, pw, c = f.shape
    nbl = hd // thl
    g = (bsz // 2) * nbl
    partials = pl.pallas_call(
        _l1_kernel,
        out_shape=jax.ShapeDtypeStruct((g, 1, c), jnp.float32),
        grid=(g,),
        in_specs=[
            pl.BlockSpec((1, thl, pw, c),
                         lambda i, nbl=nbl: (i // nbl, i % nbl, 0, 0)),
            pl.BlockSpec((1, thl, pw, c),
                         lambda i, nbl=nbl, h=bsz // 2: (h + i // nbl,
                                                         i % nbl, 0, 0)),
        ],
        out_specs=pl.BlockSpec((1, 1, c), lambda i: (i, 0, 0)),
        compiler_params=pltpu.CompilerParams(
            dimension_semantics=("parallel",), vmem_limit_bytes=_VMEM),
    )(f, f)
    n_true = (bsz // 2) * hd * wd * c
    return jnp.sum(partials) / jnp.float32(n_true)


def _pool_kernel(x_ref, o_ref, *, thp, pw, pw2, c):
    v = x_ref[0]
    rp = jnp.max(v.reshape(thp, 2, pw, c), axis=1)
    sh = jnp.concatenate([rp[:, 1:], jnp.zeros((thp, 1, c), rp.dtype)], axis=1)
    cm = jnp.maximum(rp, sh)
    ds = cm.reshape(thp, pw // 2, 2, c)[:, :, 0]
    if pw2 > pw // 2:
        zr = jnp.zeros((thp, pw2 - pw // 2, c), rp.dtype)
        ds = jnp.concatenate([ds, zr], axis=1)
    o_ref[0] = ds


def _pool(f, wd, thp=None):
    bsz, hd, pw, c = f.shape
    pw2 = _PW[wd // 2]
    if thp is None:
        thp = hd // 2
    nbp = (hd // 2) // thp
    body = functools.partial(_pool_kernel, thp=thp, pw=pw, pw2=pw2, c=c)
    return pl.pallas_call(
        body,
        out_shape=jax.ShapeDtypeStruct((bsz, hd // 2, pw2, c), jnp.bfloat16),
        grid=(bsz * nbp,),
        in_specs=[
            pl.BlockSpec((1, 2 * thp, pw, c),
                         lambda g, nbp=nbp: (g // nbp, g % nbp, 0, 0)),
        ],
        out_specs=pl.BlockSpec((1, thp, pw2, c),
                               lambda g, nbp=nbp: (g // nbp, g % nbp, 0, 0)),
        compiler_params=pltpu.CompilerParams(
            dimension_semantics=("parallel",), vmem_limit_bytes=_VMEM),
    )(f)


def kernel(inp, tgt,
           w0, b0, w1, b1, w2, b2, w3, b3, w4, b4,
           w5, b5, w6, b6, w7, b7, w8, b8, w9, b9):
    rz = _resize_norm(inp, tgt, 224)

    f = _conv1_1(rz, w0, b0)
    f, l1 = _conv_pair_l1(f, w1, b1, wd=224, th=56)
    loss = l1

    f = _pool(f, 224, thp=56)
    f = _conv(f, w2, b2, wd=112, th=112)
    f, l1 = _conv_pair_l1(f, w3, b3, wd=112, th=56)
    loss = loss + l1

    f = _pool(f, 112)
    f = _conv(f, w4, b4, wd=56, th=56)
    f = _conv(f, w5, b5, wd=56, th=56)
    f, l1 = _conv_pair_l1(f, w6, b6, wd=56, th=56)
    loss = loss + l1

    f = _pool(f, 56)
    f = _conv(f, w7, b7, wd=28, th=28)
    f = _conv(f, w8, b8, wd=28, th=28)
    f, l1 = _conv_pair_l1(f, w9, b9, wd=28, th=28)
    loss = loss + l1
    return loss
